# baseline (speedup 1.0000x reference)
import functools

import jax
import jax.numpy as jnp
from jax import lax
from jax.experimental import pallas as pl
from jax.experimental.pallas import tpu as pltpu

B = 2
P = 20000
PPAD = 20480
H = 512
W = 512
MASK_DIM = 32
HP = 128
WP = 128
NC = 100
NK = 50
G = 16
EPS = 1e-6
R64 = 64
LATENT = 128
FEAT = 36
BIGI = 2**30

_DOT = functools.partial(lax.dot_general, precision=lax.Precision.HIGHEST,
                         preferred_element_type=jnp.float32)


def _mm(a, b, ca, cb):
    return _DOT(a, b, dimension_numbers=(((ca,), (cb,)), ((), ())))


def _k1_body(c0_ref, c1_ref, comb_ref, big_ref, small_ref,
             s36_scr, conf_scr, gauss_scr, iou_scr, rs_scr):
    c0 = c0_ref[0]
    c1 = c1_ref[0]
    m = jnp.maximum(c0, c1)
    e0 = jnp.exp(c0 - m)
    e1 = jnp.exp(c1 - m)
    p = e1 / (e0 + e1)
    lin = (lax.broadcasted_iota(jnp.int32, (160, 128), 0) * 128
           + lax.broadcasted_iota(jnp.int32, (160, 128), 1))
    p = jnp.where(lin < P, p, -1.0)

    def topk_step(k, cur):
        mv = jnp.max(cur)
        cand = jnp.where(cur == mv, lin, BIGI)
        idx = jnp.min(cand)
        s36_scr[pl.ds(k, 1), :] = comb_ref[0, pl.ds(idx, 1), :]
        conf_scr[pl.ds(k, 1), :] = jnp.full((1, 128), mv, jnp.float32)
        return jnp.where(lin == idx, -2.0, cur)

    lax.fori_loop(0, NC, topk_step, p, unroll=False)

    s36 = s36_scr[:, :]
    cx = s36[:, 0:1]
    cy = s36[:, 1:2]
    sx = s36[:, 2:3] * 0.5 + 1e-2
    sy = s36[:, 3:4] * 0.5 + 1e-2
    q = lax.broadcasted_iota(jnp.int32, (1, G * G), 1)
    xs = (jnp.astype(q % G, jnp.float32) + 0.5) / G
    ys = (jnp.astype(q // G, jnp.float32) + 0.5) / G
    gauss = jnp.exp(-((xs - cx) ** 2 / (2.0 * sx * sx)
                      + (ys - cy) ** 2 / (2.0 * sy * sy)))
    gauss_scr[:, :] = gauss
    ones256 = jnp.ones((1, G * G), jnp.float32)

    def iou_step(j, carry):
        rowj = gauss_scr[pl.ds(j, 1), :]
        mn = jnp.minimum(gauss, rowj)
        mx = jnp.maximum(gauss, rowj)
        inter = _mm(ones256, mn, 1, 1)
        union = _mm(ones256, mx, 1, 1)
        iou_scr[pl.ds(j, 1), :] = inter / union
        return carry

    lax.fori_loop(0, NC, iou_step, jnp.int32(0), unroll=False)

    iou = iou_scr[:, :]
    ri = lax.broadcasted_iota(jnp.int32, (NC, NC), 0)
    cj = lax.broadcasted_iota(jnp.int32, (NC, NC), 1)
    triu = jnp.astype(ri < cj, jnp.float32)
    mm_ = iou * triu
    iou_max = jnp.max(mm_, axis=0, keepdims=True)

    e0row = jnp.astype(lax.broadcasted_iota(jnp.int32, (1, 128), 1) == 0,
                       jnp.float32)
    conf_row = _mm(e0row, conf_scr[:, :], 1, 1)
    conf_col = conf_scr[:, 0:1]
    onesr = jnp.ones((1, NC), jnp.float32)
    rs_col = _mm(mm_ * conf_row, onesr, 1, 1) * conf_col
    rs_scr[:, :] = rs_col

    lin100 = lax.broadcasted_iota(jnp.int32, (1, NC), 1)
    i8 = lax.broadcasted_iota(jnp.int32, (1, 8), 1)

    def keep_step(k, cur):
        mv = jnp.min(cur)
        cand = jnp.where(cur == mv, lin100, BIGI)
        idx = jnp.min(cand)
        big_ref[0, pl.ds(k, 1), :] = s36_scr[pl.ds(idx, 1), :]
        cv = conf_scr[pl.ds(idx, 1), 0:1]
        rv = rs_scr[pl.ds(idx, 1), 0:1]
        srow = (cv * jnp.astype(i8 == 0, jnp.float32)
                + rv * jnp.astype(i8 == 1, jnp.float32))
        small_ref[0, pl.ds(k, 1), :] = srow
        return jnp.where(lin100 == idx, jnp.float32(jnp.inf), cur)

    lax.fori_loop(0, NK, keep_step, iou_max, unroll=False)


def _stage1(c0p, c1p, comb):
    return pl.pallas_call(
        _k1_body,
        grid=(B,),
        in_specs=[
            pl.BlockSpec((1, 160, 128), lambda b: (b, 0, 0)),
            pl.BlockSpec((1, 160, 128), lambda b: (b, 0, 0)),
            pl.BlockSpec((1, P, FEAT), lambda b: (b, 0, 0)),
        ],
        out_specs=[
            pl.BlockSpec((1, NK, FEAT), lambda b: (b, 0, 0)),
            pl.BlockSpec((1, NK, 8), lambda b: (b, 0, 0)),
        ],
        out_shape=[
            jax.ShapeDtypeStruct((B, NK, FEAT), jnp.float32),
            jax.ShapeDtypeStruct((B, NK, 8), jnp.float32),
        ],
        scratch_shapes=[
            pltpu.VMEM((NC, FEAT), jnp.float32),
            pltpu.VMEM((NC, 128), jnp.float32),
            pltpu.VMEM((NC, G * G), jnp.float32),
            pltpu.VMEM((NC, NC), jnp.float32),
            pltpu.VMEM((NC, 1), jnp.float32),
        ],
    )(c0p, c1p, comb)


def _k2_body(img_ref, big_ref, small_ref, rmat_ref, rmt_ref,
             wenc_ref, wdec_ref, out_ref):
    big = big_ref[0]

    NPX = R64 * R64
    qcol = lax.broadcasted_iota(jnp.int32, (NPX, 1), 0)
    qy = qcol // R64
    qx = qcol % R64
    px = (jnp.astype(qx, jnp.float32) + 0.5) / R64
    py = (jnp.astype(qy, jnp.float32) + 0.5) / R64

    def sel_row(colmat, j, width):
        e = jnp.astype(lax.broadcasted_iota(jnp.int32, (1, width), 1) == j,
                       jnp.float32)
        return _mm(e, colmat, 1, 1)

    cxr = sel_row(big, 0, FEAT)
    cyr = sel_row(big, 1, FEAT)
    sxr = sel_row(big, 2, FEAT) * 0.5 + 1e-2
    syr = sel_row(big, 3, FEAT) * 0.5 + 1e-2
    g = jnp.exp(-((px - cxr) ** 2 / (2.0 * sxr * sxr)
                  + (py - cyr) ** 2 / (2.0 * syr * syr)))

    rs_row = sel_row(small_ref[0], 1, 8)

    ey = jnp.astype(
        lax.broadcasted_iota(jnp.int32, (NPX, R64), 0) // R64
        == lax.broadcasted_iota(jnp.int32, (NPX, R64), 1), jnp.float32)
    ex = jnp.astype(
        lax.broadcasted_iota(jnp.int32, (NPX, R64), 0) % R64
        == lax.broadcasted_iota(jnp.int32, (NPX, R64), 1), jnp.float32)

    rmat = rmat_ref[:, :]
    rmt = rmt_ref[:, :]
    ae_acc = jnp.zeros((1, NK), jnp.float32)
    latent = jnp.zeros((NK, LATENT), jnp.float32)
    fcs = []
    for c in range(3):
        a = _mm(rmat, img_ref[0, c], 1, 0)
        s = _mm(a, rmt, 1, 0)
        t = _mm(ex, s, 1, 1)
        sflat = jnp.sum(ey * t, axis=1, keepdims=True)
        fc = g * sflat
        fcs.append(fc)
        wc = wenc_ref[pl.ds(c * NPX, NPX), :]
        latent = latent + _mm(fc, wc, 0, 0)
    for c in range(3):
        wdc = wdec_ref[:, pl.ds(c * NPX, NPX)]
        recon_t = _mm(wdc, latent, 0, 1)
        d = recon_t - fcs[c]
        ae_acc = ae_acc + jnp.sum(d * d, axis=0, keepdims=True)
    ae = ae_acc * (1.0 / (3 * NPX))
    contrib = jnp.sum(ae * rs_row)
    out_ref[0] = jnp.full((8, 128), contrib, jnp.float32)


def _stage2(original, big, small, rmat, rmt, wenc, wdec):
    return pl.pallas_call(
        _k2_body,
        grid=(B,),
        in_specs=[
            pl.BlockSpec((1, 3, H, W), lambda b: (b, 0, 0, 0)),
            pl.BlockSpec((1, NK, FEAT), lambda b: (b, 0, 0)),
            pl.BlockSpec((1, NK, 8), lambda b: (b, 0, 0)),
            pl.BlockSpec((R64, H), lambda b: (0, 0)),
            pl.BlockSpec((H, R64), lambda b: (0, 0)),
            pl.BlockSpec((3 * R64 * R64, LATENT), lambda b: (0, 0)),
            pl.BlockSpec((LATENT, 3 * R64 * R64), lambda b: (0, 0)),
        ],
        out_specs=pl.BlockSpec((1, 8, 128), lambda b: (b, 0, 0)),
        out_shape=jax.ShapeDtypeStruct((B, 8, 128), jnp.float32),
    )(original, big, small, rmat, rmt, wenc, wdec)


CHUNK_ROWS = 16
NCHUNK = HP // CHUNK_ROWS
CPIX = CHUNK_ROWS * WP


def _k3_body(proto_ref, big_ref, small_ref, umat_ref, umt_ref, out_ref,
             fc_scr):
    ci = pl.program_id(1)
    proto2 = jnp.reshape(proto_ref[0], (CPIX, MASK_DIM))
    big = big_ref[0]

    def sel_row(colmat, j, width):
        e = jnp.astype(lax.broadcasted_iota(jnp.int32, (1, width), 1) == j,
                       jnp.float32)
        return _mm(e, colmat, 1, 1)

    cxr = sel_row(big, 0, FEAT)
    cyr = sel_row(big, 1, FEAT)
    sxr = sel_row(big, 2, FEAT) * 0.5 + 1e-2
    syr = sel_row(big, 3, FEAT) * 0.5 + 1e-2
    confr = sel_row(small_ref[0], 0, 8)

    mask50 = big[:, 4:36]
    asm = _mm(proto2, mask50, 1, 1)
    asm = jax.nn.sigmoid(asm)

    qcol = (lax.broadcasted_iota(jnp.int32, (CPIX, 1), 0)
            + ci * CPIX)
    qy = qcol // WP
    qx = qcol % WP
    px = (jnp.astype(qx, jnp.float32) + 0.5) / WP
    py = (jnp.astype(qy, jnp.float32) + 0.5) / HP
    ug = jnp.exp(-((px - cxr) ** 2 / (2.0 * sxr * sxr)
                   + (py - cyr) ** 2 / (2.0 * syr * syr)))

    mc = asm * ug * confr
    denom = jnp.sum(mc, axis=1, keepdims=True) + EPS
    s2 = jnp.sum(mc * mc, axis=1, keepdims=True) / denom
    fc = 1.0 - s2
    fc = jnp.where(fc != fc, 0.0, fc)

    rloc = lax.broadcasted_iota(jnp.int32, (CPIX, HP), 0)
    ccol = lax.broadcasted_iota(jnp.int32, (CPIX, HP), 1)
    ey = jnp.astype((rloc // WP + ci * CHUNK_ROWS) == ccol, jnp.float32)
    exm = jnp.astype(rloc % WP == ccol, jnp.float32)
    part = _mm(ey, fc * exm, 0, 0)

    @pl.when(ci == 0)
    def _():
        fc_scr[:, :] = part

    @pl.when(ci > 0)
    def _():
        fc_scr[:, :] = fc_scr[:, :] + part

    @pl.when(ci == NCHUNK - 1)
    def _():
        fcm = fc_scr[:, :]
        t1 = _mm(umat_ref[:, :], fcm, 1, 0)
        out_ref[0] = _mm(t1, umt_ref[:, :], 1, 0)


def _stage3(proto, big, small, umat, umt):
    return pl.pallas_call(
        _k3_body,
        grid=(B, NCHUNK),
        in_specs=[
            pl.BlockSpec((1, CHUNK_ROWS, WP, MASK_DIM),
                         lambda b, c: (b, c, 0, 0)),
            pl.BlockSpec((1, NK, FEAT), lambda b, c: (b, 0, 0)),
            pl.BlockSpec((1, NK, 8), lambda b, c: (b, 0, 0)),
            pl.BlockSpec((H, HP), lambda b, c: (0, 0)),
            pl.BlockSpec((HP, H), lambda b, c: (0, 0)),
        ],
        out_specs=pl.BlockSpec((1, H, W), lambda b, c: (b, 0, 0)),
        out_shape=jax.ShapeDtypeStruct((B, H, W), jnp.float32),
        scratch_shapes=[pltpu.VMEM((HP, HP), jnp.float32)],
    )(proto, big, small, umat, umt)


def _k4_body(img_ref, rsz_ref, out_ref):
    r0 = rsz_ref[0]
    r1 = rsz_ref[1]
    tc = r0 + r1
    acc = jnp.zeros((H, W), jnp.float32)
    for c in range(3):
        o0 = img_ref[0, c]
        o1 = img_ref[1, c]
        wm = (o0 * r0 + o1 * r1) / tc
        acc = acc + ((o0 - wm) ** 2 * r0 + (o1 - wm) ** 2 * r1) / (tc + EPS)
    total = jnp.sum(acc) / HP * B
    out_ref[:, :] = jnp.full((8, 128), total, jnp.float32)


def _stage4(original, resized):
    return pl.pallas_call(
        _k4_body,
        in_specs=[
            pl.BlockSpec((B, 3, H, W), lambda: (0, 0, 0, 0)),
            pl.BlockSpec((B, H, W), lambda: (0, 0, 0)),
        ],
        out_specs=pl.BlockSpec((8, 128), lambda: (0, 0)),
        out_shape=jax.ShapeDtypeStruct((8, 128), jnp.float32),
    )(original, resized)


def kernel(original, loc, conf, mask, proto, W_enc, W_dec):
    f32 = jnp.float32
    original = original.astype(f32)

    rmat = jax.image.resize(jnp.eye(H, dtype=f32), (R64, H), 'bilinear')
    umat = jax.image.resize(jnp.eye(HP, dtype=f32), (H, HP), 'bilinear')

    c0 = conf[:, :, 0]
    c1 = conf[:, :, 1]
    pad = ((0, 0), (0, PPAD - P))
    c0p = jnp.pad(c0, pad).reshape(B, 160, 128)
    c1p = jnp.pad(c1, pad).reshape(B, 160, 128)
    comb = jnp.concatenate([loc, mask], axis=2)

    big, small = _stage1(c0p, c1p, comb)

    ae_part = _stage2(original, big, small, rmat, rmat.T, W_enc, W_dec)
    resized = _stage3(proto, big, small, umat, umat.T)
    var_part = _stage4(original, resized)

    out_loc = big[:, :, 0:4]
    out_mask = big[:, :, 4:36]
    out_conf = small[:, :, 0]
    ae_total = (ae_part[0, 0, 0] + ae_part[1, 0, 0]) / (B * NC * NC)
    var_loss = var_part[0, 0]
    return out_loc, out_mask, out_conf, ae_total, var_loss


# speedup vs baseline: 1.1307x; 1.1307x over previous
import functools

import jax
import jax.numpy as jnp
from jax import lax
from jax.experimental import pallas as pl
from jax.experimental.pallas import tpu as pltpu

B = 2
P = 20000
PPAD = 20480
H = 512
W = 512
MASK_DIM = 32
HP = 128
WP = 128
NC = 100
NK = 50
G = 16
EPS = 1e-6
R64 = 64
LATENT = 128
FEAT = 36
BIGI = 2**30

_DOT = functools.partial(lax.dot_general, precision=lax.Precision.HIGHEST,
                         preferred_element_type=jnp.float32)


def _mm(a, b, ca, cb):
    return _DOT(a, b, dimension_numbers=(((ca,), (cb,)), ((), ())))


def _k1_body(c0_ref, c1_ref, comb_ref, big_ref, small_ref,
             s36_scr, conf_scr, gauss_scr, iou_scr, rs_scr):
    c0 = c0_ref[0]
    c1 = c1_ref[0]
    m = jnp.maximum(c0, c1)
    e0 = jnp.exp(c0 - m)
    e1 = jnp.exp(c1 - m)
    p = e1 / (e0 + e1)
    lin = (lax.broadcasted_iota(jnp.int32, (160, 128), 0) * 128
           + lax.broadcasted_iota(jnp.int32, (160, 128), 1))
    p = jnp.where(lin < P, p, -1.0)

    def topk_step(k, cur):
        mv = jnp.max(cur)
        cand = jnp.where(cur == mv, lin, BIGI)
        idx = jnp.min(cand)
        s36_scr[pl.ds(k, 1), :] = comb_ref[0, pl.ds(idx, 1), :]
        conf_scr[pl.ds(k, 1), :] = jnp.full((1, 128), mv, jnp.float32)
        return jnp.where(lin == idx, -2.0, cur)

    lax.fori_loop(0, NC, topk_step, p, unroll=False)

    s36 = s36_scr[:, :]
    cx = s36[:, 0:1]
    cy = s36[:, 1:2]
    sx = s36[:, 2:3] * 0.5 + 1e-2
    sy = s36[:, 3:4] * 0.5 + 1e-2
    q = lax.broadcasted_iota(jnp.int32, (1, G * G), 1)
    xs = (jnp.astype(q % G, jnp.float32) + 0.5) / G
    ys = (jnp.astype(q // G, jnp.float32) + 0.5) / G
    gauss = jnp.exp(-((xs - cx) ** 2 / (2.0 * sx * sx)
                      + (ys - cy) ** 2 / (2.0 * sy * sy)))
    gauss_scr[:, :] = gauss
    ones256 = jnp.ones((1, G * G), jnp.float32)

    def iou_step(j, carry):
        rowj = gauss_scr[pl.ds(j, 1), :]
        mn = jnp.minimum(gauss, rowj)
        mx = jnp.maximum(gauss, rowj)
        inter = _mm(ones256, mn, 1, 1)
        union = _mm(ones256, mx, 1, 1)
        iou_scr[pl.ds(j, 1), :] = inter / union
        return carry

    lax.fori_loop(0, NC, iou_step, jnp.int32(0), unroll=False)

    iou = iou_scr[:, :]
    ri = lax.broadcasted_iota(jnp.int32, (NC, NC), 0)
    cj = lax.broadcasted_iota(jnp.int32, (NC, NC), 1)
    triu = jnp.astype(ri < cj, jnp.float32)
    mm_ = iou * triu
    iou_max = jnp.max(mm_, axis=0, keepdims=True)

    e0row = jnp.astype(lax.broadcasted_iota(jnp.int32, (1, 128), 1) == 0,
                       jnp.float32)
    conf_row = _mm(e0row, conf_scr[:, :], 1, 1)
    conf_col = conf_scr[:, 0:1]
    onesr = jnp.ones((1, NC), jnp.float32)
    rs_col = _mm(mm_ * conf_row, onesr, 1, 1) * conf_col
    rs_scr[:, :] = rs_col

    lin100 = lax.broadcasted_iota(jnp.int32, (1, NC), 1)
    i8 = lax.broadcasted_iota(jnp.int32, (1, 8), 1)

    def keep_step(k, cur):
        mv = jnp.min(cur)
        cand = jnp.where(cur == mv, lin100, BIGI)
        idx = jnp.min(cand)
        big_ref[0, pl.ds(k, 1), :] = s36_scr[pl.ds(idx, 1), :]
        cv = conf_scr[pl.ds(idx, 1), 0:1]
        rv = rs_scr[pl.ds(idx, 1), 0:1]
        srow = (cv * jnp.astype(i8 == 0, jnp.float32)
                + rv * jnp.astype(i8 == 1, jnp.float32))
        small_ref[0, pl.ds(k, 1), :] = srow
        return jnp.where(lin100 == idx, jnp.float32(jnp.inf), cur)

    lax.fori_loop(0, NK, keep_step, iou_max, unroll=False)


def _stage1(c0p, c1p, comb):
    return pl.pallas_call(
        _k1_body,
        grid=(B,),
        in_specs=[
            pl.BlockSpec((1, 160, 128), lambda b: (b, 0, 0)),
            pl.BlockSpec((1, 160, 128), lambda b: (b, 0, 0)),
            pl.BlockSpec((1, P, FEAT), lambda b: (b, 0, 0)),
        ],
        out_specs=[
            pl.BlockSpec((1, NK, FEAT), lambda b: (b, 0, 0)),
            pl.BlockSpec((1, NK, 8), lambda b: (b, 0, 0)),
        ],
        out_shape=[
            jax.ShapeDtypeStruct((B, NK, FEAT), jnp.float32),
            jax.ShapeDtypeStruct((B, NK, 8), jnp.float32),
        ],
        scratch_shapes=[
            pltpu.VMEM((NC, FEAT), jnp.float32),
            pltpu.VMEM((NC, 128), jnp.float32),
            pltpu.VMEM((NC, G * G), jnp.float32),
            pltpu.VMEM((NC, NC), jnp.float32),
            pltpu.VMEM((NC, 1), jnp.float32),
        ],
    )(c0p, c1p, comb)


def _k2_body(img_ref, big_ref, small_ref, rmat_ref, rmt_ref,
             wenc_ref, wdec_ref, out_ref):
    big = big_ref[0]

    NPX = R64 * R64
    qcol = lax.broadcasted_iota(jnp.int32, (NPX, 1), 0)
    qy = qcol // R64
    qx = qcol % R64
    px = (jnp.astype(qx, jnp.float32) + 0.5) / R64
    py = (jnp.astype(qy, jnp.float32) + 0.5) / R64

    def sel_row(colmat, j, width):
        e = jnp.astype(lax.broadcasted_iota(jnp.int32, (1, width), 1) == j,
                       jnp.float32)
        return _mm(e, colmat, 1, 1)

    cxr = sel_row(big, 0, FEAT)
    cyr = sel_row(big, 1, FEAT)
    sxr = sel_row(big, 2, FEAT) * 0.5 + 1e-2
    syr = sel_row(big, 3, FEAT) * 0.5 + 1e-2
    g = jnp.exp(-((px - cxr) ** 2 / (2.0 * sxr * sxr)
                  + (py - cyr) ** 2 / (2.0 * syr * syr)))

    rs_row = sel_row(small_ref[0], 1, 8)

    ey = jnp.astype(
        lax.broadcasted_iota(jnp.int32, (NPX, R64), 0) // R64
        == lax.broadcasted_iota(jnp.int32, (NPX, R64), 1), jnp.float32)
    ex = jnp.astype(
        lax.broadcasted_iota(jnp.int32, (NPX, R64), 0) % R64
        == lax.broadcasted_iota(jnp.int32, (NPX, R64), 1), jnp.float32)

    rmat = rmat_ref[:, :]
    rmt = rmt_ref[:, :]
    ae_acc = jnp.zeros((1, NK), jnp.float32)
    latent = jnp.zeros((NK, LATENT), jnp.float32)
    fcs = []
    for c in range(3):
        a = _mm(rmat, img_ref[0, c], 1, 0)
        s = _mm(a, rmt, 1, 0)
        t = _mm(ex, s, 1, 1)
        sflat = jnp.sum(ey * t, axis=1, keepdims=True)
        fc = g * sflat
        fcs.append(fc)
        wc = wenc_ref[pl.ds(c * NPX, NPX), :]
        latent = latent + _mm(fc, wc, 0, 0)
    for c in range(3):
        wdc = wdec_ref[:, pl.ds(c * NPX, NPX)]
        recon_t = _mm(wdc, latent, 0, 1)
        d = recon_t - fcs[c]
        ae_acc = ae_acc + jnp.sum(d * d, axis=0, keepdims=True)
    ae = ae_acc * (1.0 / (3 * NPX))
    contrib = jnp.sum(ae * rs_row)
    out_ref[0] = jnp.full((8, 128), contrib, jnp.float32)


def _stage2(original, big, small, rmat, rmt, wenc, wdec):
    return pl.pallas_call(
        _k2_body,
        grid=(B,),
        in_specs=[
            pl.BlockSpec((1, 3, H, W), lambda b: (b, 0, 0, 0)),
            pl.BlockSpec((1, NK, FEAT), lambda b: (b, 0, 0)),
            pl.BlockSpec((1, NK, 8), lambda b: (b, 0, 0)),
            pl.BlockSpec((R64, H), lambda b: (0, 0)),
            pl.BlockSpec((H, R64), lambda b: (0, 0)),
            pl.BlockSpec((3 * R64 * R64, LATENT), lambda b: (0, 0)),
            pl.BlockSpec((LATENT, 3 * R64 * R64), lambda b: (0, 0)),
        ],
        out_specs=pl.BlockSpec((1, 8, 128), lambda b: (b, 0, 0)),
        out_shape=jax.ShapeDtypeStruct((B, 8, 128), jnp.float32),
    )(original, big, small, rmat, rmt, wenc, wdec)


CHUNK_ROWS = 16
NCHUNK = HP // CHUNK_ROWS
CPIX = CHUNK_ROWS * WP


def _k3_body(proto_ref, big_ref, small_ref, umat_ref, umt_ref, out_ref,
             fc_scr):
    ci = pl.program_id(1)
    proto2 = jnp.reshape(proto_ref[0], (CPIX, MASK_DIM))
    big = big_ref[0]

    cx = big[:, 0:1]
    cy = big[:, 1:2]
    sx = big[:, 2:3] * 0.5 + 1e-2
    sy = big[:, 3:4] * 0.5 + 1e-2
    confc = small_ref[0][:, 0:1]

    mask50 = big[:, 4:36]
    asm = _mm(mask50, proto2, 1, 1)
    asm = jax.nn.sigmoid(asm)

    qrow = (lax.broadcasted_iota(jnp.int32, (1, CPIX), 1)
            + ci * CPIX)
    qy = qrow // WP
    qx = qrow % WP
    px = (jnp.astype(qx, jnp.float32) + 0.5) / WP
    py = (jnp.astype(qy, jnp.float32) + 0.5) / HP
    ug = jnp.exp(-((px - cx) ** 2 / (2.0 * sx * sx)
                   + (py - cy) ** 2 / (2.0 * sy * sy)))

    mc = asm * ug * confc
    denom = jnp.sum(mc, axis=0, keepdims=True) + EPS
    s2 = jnp.sum(mc * mc, axis=0, keepdims=True) / denom
    fc = 1.0 - s2
    fc = jnp.where(fc != fc, 0.0, fc)

    for j in range(CHUNK_ROWS):
        fc_scr[pl.ds(ci * CHUNK_ROWS + j, 1), :] = (
            fc[:, j * WP:(j + 1) * WP])

    @pl.when(ci == NCHUNK - 1)
    def _():
        fcm = fc_scr[:, :]
        t1 = _mm(umat_ref[:, :], fcm, 1, 0)
        out_ref[0] = _mm(t1, umt_ref[:, :], 1, 0)


def _stage3(proto, big, small, umat, umt):
    return pl.pallas_call(
        _k3_body,
        grid=(B, NCHUNK),
        in_specs=[
            pl.BlockSpec((1, CHUNK_ROWS, WP, MASK_DIM),
                         lambda b, c: (b, c, 0, 0)),
            pl.BlockSpec((1, NK, FEAT), lambda b, c: (b, 0, 0)),
            pl.BlockSpec((1, NK, 8), lambda b, c: (b, 0, 0)),
            pl.BlockSpec((H, HP), lambda b, c: (0, 0)),
            pl.BlockSpec((HP, H), lambda b, c: (0, 0)),
        ],
        out_specs=pl.BlockSpec((1, H, W), lambda b, c: (b, 0, 0)),
        out_shape=jax.ShapeDtypeStruct((B, H, W), jnp.float32),
        scratch_shapes=[pltpu.VMEM((HP, HP), jnp.float32)],
    )(proto, big, small, umat, umt)


def _k4_body(img_ref, rsz_ref, out_ref):
    r0 = rsz_ref[0]
    r1 = rsz_ref[1]
    tc = r0 + r1
    acc = jnp.zeros((H, W), jnp.float32)
    for c in range(3):
        o0 = img_ref[0, c]
        o1 = img_ref[1, c]
        wm = (o0 * r0 + o1 * r1) / tc
        acc = acc + ((o0 - wm) ** 2 * r0 + (o1 - wm) ** 2 * r1) / (tc + EPS)
    total = jnp.sum(acc) / HP * B
    out_ref[:, :] = jnp.full((8, 128), total, jnp.float32)


def _stage4(original, resized):
    return pl.pallas_call(
        _k4_body,
        in_specs=[
            pl.BlockSpec((B, 3, H, W), lambda: (0, 0, 0, 0)),
            pl.BlockSpec((B, H, W), lambda: (0, 0, 0)),
        ],
        out_specs=pl.BlockSpec((8, 128), lambda: (0, 0)),
        out_shape=jax.ShapeDtypeStruct((8, 128), jnp.float32),
    )(original, resized)


def kernel(original, loc, conf, mask, proto, W_enc, W_dec):
    f32 = jnp.float32
    original = original.astype(f32)

    rmat = jax.image.resize(jnp.eye(H, dtype=f32), (R64, H), 'bilinear')
    umat = jax.image.resize(jnp.eye(HP, dtype=f32), (H, HP), 'bilinear')

    c0 = conf[:, :, 0]
    c1 = conf[:, :, 1]
    pad = ((0, 0), (0, PPAD - P))
    c0p = jnp.pad(c0, pad).reshape(B, 160, 128)
    c1p = jnp.pad(c1, pad).reshape(B, 160, 128)
    comb = jnp.concatenate([loc, mask], axis=2)

    big, small = _stage1(c0p, c1p, comb)

    ae_part = _stage2(original, big, small, rmat, rmat.T, W_enc, W_dec)
    resized = _stage3(proto, big, small, umat, umat.T)
    var_part = _stage4(original, resized)

    out_loc = big[:, :, 0:4]
    out_mask = big[:, :, 4:36]
    out_conf = small[:, :, 0]
    ae_total = (ae_part[0, 0, 0] + ae_part[1, 0, 0]) / (B * NC * NC)
    var_loss = var_part[0, 0]
    return out_loc, out_mask, out_conf, ae_total, var_loss


# speedup vs baseline: 1.2251x; 1.0835x over previous
import functools

import jax
import jax.numpy as jnp
from jax import lax
from jax.experimental import pallas as pl
from jax.experimental.pallas import tpu as pltpu
from jax.experimental.pallas import tpu_sc as plsc

B = 2
P = 20000
PPAD = 20480
H = 512
W = 512
MASK_DIM = 32
HP = 128
WP = 128
NC = 100
NK = 50
G = 16
EPS = 1e-6
R64 = 64
LATENT = 128
FEAT = 36
BIGI = 2**30

NW = 32
PERW = PPAD // NW
NCHK = PERW // 16
CAND = 112
COMBW = 128
SLO = -0x7F800001
SHI = 0x7F800001

_DOT = functools.partial(lax.dot_general, precision=lax.Precision.HIGHEST,
                         preferred_element_type=jnp.float32)


def _mm(a, b, ca, cb):
    return _DOT(a, b, dimension_numbers=(((ca,), (cb,)), ((), ())))


_DOTD = functools.partial(lax.dot_general, precision=lax.Precision.DEFAULT,
                          preferred_element_type=jnp.float32)


def _mmd(a, b, ca, cb):
    return _DOTD(a, b, dimension_numbers=(((ca,), (cb,)), ((), ())))


def _k0_body(c0_ref, c1_ref, sidx_ref, sval_ref):
    c0 = c0_ref[0]
    c1 = c1_ref[0]
    m = jnp.maximum(c0, c1)
    e0 = jnp.exp(c0 - m)
    e1 = jnp.exp(c1 - m)
    p = e1 / (e0 + e1)
    lin = (lax.broadcasted_iota(jnp.int32, (160, 128), 0) * 128
           + lax.broadcasted_iota(jnp.int32, (160, 128), 1))
    p = jnp.where(lin < P, p, -1.0)
    lane128 = lax.broadcasted_iota(jnp.int32, (1, 128), 1)

    def step(k, carry):
        cur, idxrow, valrow = carry
        mv = jnp.max(cur)
        pidx = jnp.min(jnp.where(cur == mv, lin, BIGI))
        valrow = jnp.where(lane128 == k, mv, valrow)
        idxrow = jnp.where(lane128 == k, pidx, idxrow)
        return (jnp.where(lin == pidx, -2.0, cur), idxrow, valrow)

    _, idxrow, valrow = lax.fori_loop(
        0, NC, step,
        (p, jnp.zeros((1, 128), jnp.int32), jnp.full((1, 128), -1.0)),
        unroll=False)
    sidx_ref[0] = idxrow
    sval_ref[0] = valrow


def _stage0(c0p, c1p):
    return pl.pallas_call(
        _k0_body,
        grid=(B,),
        in_specs=[
            pl.BlockSpec((1, 160, 128), lambda b: (b, 0, 0)),
            pl.BlockSpec((1, 160, 128), lambda b: (b, 0, 0)),
        ],
        out_specs=[
            pl.BlockSpec((1, 1, 128), lambda b: (b, 0, 0)),
            pl.BlockSpec((1, 1, 128), lambda b: (b, 0, 0)),
        ],
        out_shape=[
            jax.ShapeDtypeStruct((B, 1, 128), jnp.int32),
            jax.ShapeDtypeStruct((B, 1, 128), jnp.float32),
        ],
    )(c0p, c1p)


def _sc_gather_body(sidx_hbm, comb_hbm, rows_hbm, idx_v, rows_v, sem):
    cid = lax.axis_index("c")
    sid = lax.axis_index("s")
    wid = sid * 2 + cid
    for b in range(B):
        @pl.when(wid == b)
        def _g(b=b):
            pltpu.sync_copy(sidx_hbm.at[b], idx_v)
            for j in range(8):
                off = j * 16
                idx_v[pl.ds(off, 16)] = idx_v[pl.ds(off, 16)] + b * P
            pltpu.async_copy(comb_hbm.at[idx_v], rows_v, sem).wait()
            pltpu.sync_copy(rows_v, rows_hbm.at[b])


def _sc_gather(sidx, combflat):
    mesh = plsc.VectorSubcoreMesh(core_axis_name="c", subcore_axis_name="s",
                                  num_cores=2, num_subcores=16)
    f = pl.kernel(
        _sc_gather_body,
        out_type=jax.ShapeDtypeStruct((B, 128, COMBW), jnp.float32),
        mesh=mesh,
        scratch_types=[
            pltpu.VMEM((128,), jnp.int32),
            pltpu.VMEM((128, COMBW), jnp.float32),
            pltpu.SemaphoreType.DMA,
        ],
    )
    return f(sidx, combflat)


def _k1b_body(rows_ref, sval_ref, big_ref, small_ref,
              conf_scr, gauss_scr, iou_scr, rs_scr):
    rows = rows_ref[0]
    s36 = rows[0:NC, 0:FEAT]
    sval = sval_ref[0]
    e100 = jnp.astype(
        lax.broadcasted_iota(jnp.int32, (NC, 128), 0)
        == lax.broadcasted_iota(jnp.int32, (NC, 128), 1), jnp.float32)
    conf_col = _mm(e100, sval, 1, 1)
    conf_scr[:, :] = conf_col
    conf_row = sval[:, 0:NC]

    cx = s36[:, 0:1]
    cy = s36[:, 1:2]
    sx = s36[:, 2:3] * 0.5 + 1e-2
    sy = s36[:, 3:4] * 0.5 + 1e-2
    q = lax.broadcasted_iota(jnp.int32, (1, G * G), 1)
    xs = (jnp.astype(q % G, jnp.float32) + 0.5) / G
    ys = (jnp.astype(q // G, jnp.float32) + 0.5) / G
    gauss = jnp.exp(-((xs - cx) ** 2 / (2.0 * sx * sx)
                      + (ys - cy) ** 2 / (2.0 * sy * sy)))
    gauss_scr[:, :] = gauss
    ones256 = jnp.ones((1, G * G), jnp.float32)

    def iou_step(j, carry):
        rowj = gauss_scr[pl.ds(j, 1), :]
        mn = jnp.minimum(gauss, rowj)
        mx = jnp.maximum(gauss, rowj)
        inter = _mm(ones256, mn, 1, 1)
        union = _mm(ones256, mx, 1, 1)
        iou_scr[pl.ds(j, 1), :] = inter / union
        return carry

    lax.fori_loop(0, NC, iou_step, jnp.int32(0), unroll=False)

    iou = iou_scr[:, :]
    ri = lax.broadcasted_iota(jnp.int32, (NC, NC), 0)
    cj = lax.broadcasted_iota(jnp.int32, (NC, NC), 1)
    triu = jnp.astype(ri < cj, jnp.float32)
    mm_ = iou * triu
    iou_max = jnp.max(mm_, axis=0, keepdims=True)

    onesr = jnp.ones((1, NC), jnp.float32)
    rs_col = _mm(mm_ * conf_row, onesr, 1, 1) * conf_col
    rs_scr[:, :] = rs_col

    lin100 = lax.broadcasted_iota(jnp.int32, (1, NC), 1)
    i8 = lax.broadcasted_iota(jnp.int32, (1, 8), 1)

    def keep_step(k, cur):
        mv = jnp.min(cur)
        cand = jnp.where(cur == mv, lin100, BIGI)
        idx = jnp.min(cand)
        big_ref[0, pl.ds(k, 1), :] = rows_ref[0, pl.ds(idx, 1), 0:FEAT]
        cv = conf_scr[pl.ds(idx, 1), 0:1]
        rv = rs_scr[pl.ds(idx, 1), 0:1]
        srow = (cv * jnp.astype(i8 == 0, jnp.float32)
                + rv * jnp.astype(i8 == 1, jnp.float32))
        small_ref[0, pl.ds(k, 1), :] = srow
        return jnp.where(lin100 == idx, jnp.float32(jnp.inf), cur)

    lax.fori_loop(0, NK, keep_step, iou_max, unroll=False)


def _stage1b(rows, svals3):
    return pl.pallas_call(
        _k1b_body,
        grid=(B,),
        in_specs=[
            pl.BlockSpec((1, 128, COMBW), lambda b: (b, 0, 0)),
            pl.BlockSpec((1, 1, 128), lambda b: (b, 0, 0)),
        ],
        out_specs=[
            pl.BlockSpec((1, NK, FEAT), lambda b: (b, 0, 0)),
            pl.BlockSpec((1, NK, 8), lambda b: (b, 0, 0)),
        ],
        out_shape=[
            jax.ShapeDtypeStruct((B, NK, FEAT), jnp.float32),
            jax.ShapeDtypeStruct((B, NK, 8), jnp.float32),
        ],
        scratch_shapes=[
            pltpu.VMEM((NC, 1), jnp.float32),
            pltpu.VMEM((NC, G * G), jnp.float32),
            pltpu.VMEM((NC, NC), jnp.float32),
            pltpu.VMEM((NC, 1), jnp.float32),
        ],
    )(rows, svals3)


def _k2_body(img_ref, big_ref, small_ref, rmat_ref, rmt_ref,
             wenc_ref, wdec_ref, out_ref):
    big = big_ref[0]

    NPX = R64 * R64
    qcol = lax.broadcasted_iota(jnp.int32, (NPX, 1), 0)
    qy = qcol // R64
    qx = qcol % R64
    px = (jnp.astype(qx, jnp.float32) + 0.5) / R64
    py = (jnp.astype(qy, jnp.float32) + 0.5) / R64

    def sel_row(colmat, j, width):
        e = jnp.astype(lax.broadcasted_iota(jnp.int32, (1, width), 1) == j,
                       jnp.float32)
        return _mm(e, colmat, 1, 1)

    cxr = sel_row(big, 0, FEAT)
    cyr = sel_row(big, 1, FEAT)
    sxr = sel_row(big, 2, FEAT) * 0.5 + 1e-2
    syr = sel_row(big, 3, FEAT) * 0.5 + 1e-2
    g = jnp.exp(-((px - cxr) ** 2 / (2.0 * sxr * sxr)
                  + (py - cyr) ** 2 / (2.0 * syr * syr)))

    rs_row = sel_row(small_ref[0], 1, 8)

    ey = jnp.astype(
        lax.broadcasted_iota(jnp.int32, (NPX, R64), 0) // R64
        == lax.broadcasted_iota(jnp.int32, (NPX, R64), 1), jnp.float32)
    ex = jnp.astype(
        lax.broadcasted_iota(jnp.int32, (NPX, R64), 0) % R64
        == lax.broadcasted_iota(jnp.int32, (NPX, R64), 1), jnp.float32)

    rmat = rmat_ref[:, :]
    rmt = rmt_ref[:, :]
    ae_acc = jnp.zeros((1, NK), jnp.float32)
    latent = jnp.zeros((NK, LATENT), jnp.float32)
    fcs = []
    for c in range(3):
        a = _mmd(rmat, img_ref[0, c], 1, 0)
        s = _mmd(a, rmt, 1, 0)
        t = _mmd(ex, s, 1, 1)
        sflat = jnp.sum(ey * t, axis=1, keepdims=True)
        fc = g * sflat
        fcs.append(fc)
        wc = wenc_ref[pl.ds(c * NPX, NPX), :]
        latent = latent + _mmd(fc, wc, 0, 0)
    for c in range(3):
        wdc = wdec_ref[:, pl.ds(c * NPX, NPX)]
        recon_t = _mmd(wdc, latent, 0, 1)
        d = recon_t - fcs[c]
        ae_acc = ae_acc + jnp.sum(d * d, axis=0, keepdims=True)
    ae = ae_acc * (1.0 / (3 * NPX))
    contrib = jnp.sum(ae * rs_row)
    out_ref[0] = jnp.full((8, 128), contrib, jnp.float32)


def _stage2(original, big, small, rmat, rmt, wenc, wdec):
    return pl.pallas_call(
        _k2_body,
        grid=(B,),
        in_specs=[
            pl.BlockSpec((1, 3, H, W), lambda b: (b, 0, 0, 0)),
            pl.BlockSpec((1, NK, FEAT), lambda b: (b, 0, 0)),
            pl.BlockSpec((1, NK, 8), lambda b: (b, 0, 0)),
            pl.BlockSpec((R64, H), lambda b: (0, 0)),
            pl.BlockSpec((H, R64), lambda b: (0, 0)),
            pl.BlockSpec((3 * R64 * R64, LATENT), lambda b: (0, 0)),
            pl.BlockSpec((LATENT, 3 * R64 * R64), lambda b: (0, 0)),
        ],
        out_specs=pl.BlockSpec((1, 8, 128), lambda b: (b, 0, 0)),
        out_shape=jax.ShapeDtypeStruct((B, 8, 128), jnp.float32),
    )(original, big, small, rmat, rmt, wenc, wdec)


CHUNK_ROWS = 16
NCHUNK = HP // CHUNK_ROWS
CPIX = CHUNK_ROWS * WP


def _k3_body(proto_ref, big_ref, small_ref, umat_ref, umt_ref, out_ref,
             fc_scr):
    ci = pl.program_id(1)
    proto2 = jnp.reshape(proto_ref[0], (CPIX, MASK_DIM))
    big = big_ref[0]

    cx = big[:, 0:1]
    cy = big[:, 1:2]
    sx = big[:, 2:3] * 0.5 + 1e-2
    sy = big[:, 3:4] * 0.5 + 1e-2
    confc = small_ref[0][:, 0:1]

    mask50 = big[:, 4:36]
    asm = _mmd(mask50, proto2, 1, 1)
    asm = jax.nn.sigmoid(asm)

    qrow = (lax.broadcasted_iota(jnp.int32, (1, CPIX), 1)
            + ci * CPIX)
    qy = qrow // WP
    qx = qrow % WP
    px = (jnp.astype(qx, jnp.float32) + 0.5) / WP
    py = (jnp.astype(qy, jnp.float32) + 0.5) / HP
    ug = jnp.exp(-((px - cx) ** 2 / (2.0 * sx * sx)
                   + (py - cy) ** 2 / (2.0 * sy * sy)))

    mc = asm * ug * confc
    denom = jnp.sum(mc, axis=0, keepdims=True) + EPS
    s2 = jnp.sum(mc * mc, axis=0, keepdims=True) / denom
    fc = 1.0 - s2
    fc = jnp.where(fc != fc, 0.0, fc)

    for j in range(CHUNK_ROWS):
        fc_scr[pl.ds(ci * CHUNK_ROWS + j, 1), :] = (
            fc[:, j * WP:(j + 1) * WP])

    @pl.when(ci == NCHUNK - 1)
    def _():
        fcm = fc_scr[:, :]
        t1 = _mmd(umat_ref[:, :], fcm, 1, 0)
        out_ref[0] = _mmd(t1, umt_ref[:, :], 1, 0)


def _stage3(proto, big, small, umat, umt):
    return pl.pallas_call(
        _k3_body,
        grid=(B, NCHUNK),
        in_specs=[
            pl.BlockSpec((1, CHUNK_ROWS, WP, MASK_DIM),
                         lambda b, c: (b, c, 0, 0)),
            pl.BlockSpec((1, NK, FEAT), lambda b, c: (b, 0, 0)),
            pl.BlockSpec((1, NK, 8), lambda b, c: (b, 0, 0)),
            pl.BlockSpec((H, HP), lambda b, c: (0, 0)),
            pl.BlockSpec((HP, H), lambda b, c: (0, 0)),
        ],
        out_specs=pl.BlockSpec((1, H, W), lambda b, c: (b, 0, 0)),
        out_shape=jax.ShapeDtypeStruct((B, H, W), jnp.float32),
        scratch_shapes=[pltpu.VMEM((HP, HP), jnp.float32)],
    )(proto, big, small, umat, umt)


def _k4_body(img_ref, rsz_ref, out_ref):
    r0 = rsz_ref[0]
    r1 = rsz_ref[1]
    tc = r0 + r1
    acc = jnp.zeros((H, W), jnp.float32)
    for c in range(3):
        o0 = img_ref[0, c]
        o1 = img_ref[1, c]
        wm = (o0 * r0 + o1 * r1) / tc
        acc = acc + ((o0 - wm) ** 2 * r0 + (o1 - wm) ** 2 * r1) / (tc + EPS)
    total = jnp.sum(acc) / HP * B
    out_ref[:, :] = jnp.full((8, 128), total, jnp.float32)


def _stage4(original, resized):
    return pl.pallas_call(
        _k4_body,
        in_specs=[
            pl.BlockSpec((B, 3, H, W), lambda: (0, 0, 0, 0)),
            pl.BlockSpec((B, H, W), lambda: (0, 0, 0)),
        ],
        out_specs=pl.BlockSpec((8, 128), lambda: (0, 0)),
        out_shape=jax.ShapeDtypeStruct((8, 128), jnp.float32),
    )(original, resized)


def kernel(original, loc, conf, mask, proto, W_enc, W_dec):
    f32 = jnp.float32
    original = original.astype(f32)

    rmat = jax.image.resize(jnp.eye(H, dtype=f32), (R64, H), 'bilinear')
    umat = jax.image.resize(jnp.eye(HP, dtype=f32), (H, HP), 'bilinear')

    pad = ((0, 0), (0, PPAD - P))
    c0p = jnp.pad(conf[:, :, 0], pad).reshape(B, 160, 128)
    c1p = jnp.pad(conf[:, :, 1], pad).reshape(B, 160, 128)
    combflat = jnp.concatenate(
        [loc, mask, jnp.zeros((B, P, COMBW - FEAT), f32)], axis=2)
    combflat = combflat.reshape(B * P, COMBW)

    sidx3, svals3 = _stage0(c0p, c1p)
    rows = _sc_gather(sidx3.reshape(B, 128), combflat)
    big, small = _stage1b(rows, svals3)

    ae_part = _stage2(original, big, small, rmat, rmat.T, W_enc, W_dec)
    resized = _stage3(proto, big, small, umat, umat.T)
    var_part = _stage4(original, resized)

    out_loc = big[:, :, 0:4]
    out_mask = big[:, :, 4:36]
    out_conf = small[:, :, 0]
    ae_total = (ae_part[0, 0, 0] + ae_part[1, 0, 0]) / (B * NC * NC)
    var_loss = var_part[0, 0]
    return out_loc, out_mask, out_conf, ae_total, var_loss
